# Initial kernel scaffold; baseline (speedup 1.0000x reference)
import jax
import jax.numpy as jnp
from jax.experimental import pallas as pl


def kernel(x, src, dst, neg_dst, batch_msg, emb0, emb1, emb2, emb3, emb4, emb5, emb6, emb7, emb8, emb9, edge_W, edge_b, out_W, out_b):
    raise NotImplementedError("write your pallas kernel here")


# speedup vs baseline: 1.3792x; 1.3792x over previous
import functools

import jax
import jax.numpy as jnp
from jax import lax
from jax.experimental import pallas as pl
from jax.experimental.pallas import tpu as pltpu
from jax.experimental.pallas import tpu_sc as plsc

_D = 64
_B = 16384
_R = 3 * _B
_NC, _NS, _L = 2, 16, 16
_NW = _NC * _NS
_RPW = _R // _NW
_C = 128
_NCH = _RPW // _C


@functools.partial(
    pl.kernel,
    mesh=plsc.VectorSubcoreMesh(core_axis_name="c", subcore_axis_name="s"),
    out_type=jax.ShapeDtypeStruct((_R, _D), jnp.float32),
    compiler_params=pltpu.CompilerParams(use_tc_tiling_on_sc=False),
    scratch_types=[
        pltpu.VMEM((_C,), jnp.int32),
        pltpu.VMEM((9, _C), jnp.int32),
        pltpu.VMEM((_C + _L,), jnp.int32),
        pltpu.VMEM((9, _C, _D), jnp.float32),
        pltpu.VMEM((_C, _D), jnp.float32),
        pltpu.VMEM((3 * _D,), jnp.float32),
        pltpu.SemaphoreType.DMA,
    ],
)
def _sc_encode(xc0, xc1, xc2, xc3, xc4, xc5, xc6, xc7, xc8, xc9,
               idx_all, e0, e1, e2, e3, e4, e5, e6, e7, e8, e9,
               hout, idx_v, cols, tlist, rbuf, hloc, emb0v, sem):
    embs = (e1, e2, e3, e4, e5, e6, e7, e8, e9)
    xcs = (xc1, xc2, xc3, xc4, xc5, xc6, xc7, xc8, xc9)
    wid = lax.axis_index("s") * _NC + lax.axis_index("c")
    pltpu.sync_copy(e0, emb0v)
    e0sl = [[emb0v[pl.ds(t * _D + dv * _L, _L)] for dv in range(_D // _L)]
            for t in range(3)]
    base_w = wid * _RPW

    def chunk(ci, carry):
        base = base_w + ci * _C
        pltpu.sync_copy(idx_all.at[pl.ds(base, _C)], idx_v)
        xcps = [pltpu.async_copy(xc0.at[idx_v], tlist.at[pl.ds(0, _C)], sem)]
        xcps += [pltpu.async_copy(xcs[f].at[idx_v], cols.at[f], sem)
                 for f in range(9)]
        for cp in xcps:
            cp.wait()
        cps = [pltpu.async_copy(embs[f].at[cols.at[f]], rbuf.at[f], sem)
               for f in range(9)]
        for cp in cps:
            cp.wait()

        def row(r, c2):
            t = tlist[pl.ds(r, _L)][0]
            s0 = jnp.where(t == 0, 1.0, 0.0)
            s1 = jnp.where(t == 1, 1.0, 0.0)
            s2 = jnp.where(t == 2, 1.0, 0.0)
            b0 = lax.broadcast(s0, (_L,))
            b1 = lax.broadcast(s1, (_L,))
            b2 = lax.broadcast(s2, (_L,))
            for dv in range(_D // _L):
                sl = pl.ds(dv * _L, _L)
                h0 = (e0sl[0][dv] * b0 + e0sl[1][dv] * b1 + e0sl[2][dv] * b2)
                a = (rbuf[0, r, sl] + rbuf[1, r, sl]
                     + rbuf[2, r, sl] + rbuf[3, r, sl])
                b = rbuf[4, r, sl]
                c = (rbuf[5, r, sl] + rbuf[6, r, sl]
                     + rbuf[7, r, sl] + rbuf[8, r, sl])
                hloc[r, sl] = h0 + a * b0 + b * b1 + c * b2
            return c2

        lax.fori_loop(0, _C, row, 0)
        pltpu.sync_copy(hloc, hout.at[pl.ds(base, _C)])
        return carry

    lax.fori_loop(0, _NCH, chunk, 0)


_BLK = 2048
_NB = _B // _BLK


def _tc_epilogue(h, bm, ew, eb, ow, ob):
    def body(hs, hp, hn, bmr, ewr, ebr, owr, obr, opos, oneg):
        w = owr[...]
        u = jnp.sum(w.T * ewr[...], axis=0, keepdims=True)
        const = jnp.sum(ebr[...] * w[0]) + obr[...][0]
        ms = jnp.sum(bmr[...] * u, axis=1, keepdims=True) + const
        hs_, hp_, hn_ = hs[...], hp[...], hn[...]
        pos = jnp.maximum(hs_ + hp_, 0.0)
        neg = jnp.maximum(hs_ + hn_, 0.0)
        opos[...] = jnp.sum(pos * w, axis=1, keepdims=True) + ms
        oneg[...] = jnp.sum(neg * w, axis=1, keepdims=True) + ms

    return pl.pallas_call(
        body,
        grid=(_NB,),
        in_specs=[
            pl.BlockSpec((_BLK, _D), lambda i: (i, 0)),
            pl.BlockSpec((_BLK, _D), lambda i: (i + _NB, 0)),
            pl.BlockSpec((_BLK, _D), lambda i: (i + 2 * _NB, 0)),
            pl.BlockSpec((_BLK, 27), lambda i: (i, 0)),
            pl.BlockSpec((_D, 27), lambda i: (0, 0)),
            pl.BlockSpec((_D,), lambda i: (0,)),
            pl.BlockSpec((1, _D), lambda i: (0, 0)),
            pl.BlockSpec((1,), lambda i: (0,)),
        ],
        out_specs=[
            pl.BlockSpec((_BLK, 1), lambda i: (i, 0)),
            pl.BlockSpec((_BLK, 1), lambda i: (i, 0)),
        ],
        out_shape=[
            jax.ShapeDtypeStruct((_B, 1), jnp.float32),
            jax.ShapeDtypeStruct((_B, 1), jnp.float32),
        ],
    )(h, h, h, bm, ew, eb, ow, ob)


def kernel(x, src, dst, neg_dst, batch_msg,
           emb0, emb1, emb2, emb3, emb4, emb5, emb6, emb7, emb8, emb9,
           edge_W, edge_b, out_W, out_b):
    xt = x.T
    xcs = [xt[f] for f in range(10)]
    idx_all = jnp.concatenate([src, dst, neg_dst], axis=0)
    h = _sc_encode(*xcs, idx_all, emb0.reshape(-1), emb1, emb2, emb3, emb4,
                   emb5, emb6, emb7, emb8, emb9)
    out_pos, out_neg = _tc_epilogue(h, batch_msg, edge_W, edge_b, out_W, out_b)
    return (out_pos, out_neg)
